# baseline (speedup 1.0000x reference)
import functools

import jax
import jax.numpy as jnp
import numpy as np
from jax.experimental import pallas as pl
from jax.experimental.pallas import tpu as pltpu

HID = 128
INV_SQRT_HID = 1.0 / np.sqrt(128.0)


def _decoder_body(z_ref, w1, b1, w2, b2, w3, b3, w4, b4, out_ref):
    z = z_ref[...]
    h = jnp.maximum(jnp.dot(z, w1[...], preferred_element_type=jnp.float32) + b1[...], 0.0)
    h = jnp.maximum(jnp.dot(h, w2[...], preferred_element_type=jnp.float32) + b2[...], 0.0)
    h = jnp.maximum(jnp.dot(h, w3[...], preferred_element_type=jnp.float32) + b3[...], 0.0)
    out_ref[...] = jnp.dot(h, w4[...], preferred_element_type=jnp.float32) + b4[...]


def _decoder_mlp(z, p1, p2, p3, p4):
    n = z.shape[0]
    blk = 2000
    grid = n // blk
    w4p = jnp.zeros((64, 128), jnp.float32).at[:, :5].set(p4["w"])
    b4p = jnp.zeros((128,), jnp.float32).at[:5].set(p4["b"])
    out = pl.pallas_call(
        _decoder_body,
        grid=(grid,),
        in_specs=[
            pl.BlockSpec((blk, 256), lambda i: (i, 0)),
            pl.BlockSpec((256, 128), lambda i: (0, 0)),
            pl.BlockSpec((128,), lambda i: (0,)),
            pl.BlockSpec((128, 256), lambda i: (0, 0)),
            pl.BlockSpec((256,), lambda i: (0,)),
            pl.BlockSpec((256, 64), lambda i: (0, 0)),
            pl.BlockSpec((64,), lambda i: (0,)),
            pl.BlockSpec((64, 128), lambda i: (0, 0)),
            pl.BlockSpec((128,), lambda i: (0,)),
        ],
        out_specs=pl.BlockSpec((blk, 128), lambda i: (i, 0)),
        out_shape=jax.ShapeDtypeStruct((n, 128), jnp.float32),
    )(z, p1["w"], p1["b"], p2["w"], p2["b"], p3["w"], p3["b"], w4p, b4p)
    return out[:, :5]


def _linear(x, p):
    return x @ p["w"] + p["b"]


def _transformer_conv(x_src, x_dst, edge_index, edge_attr, p):
    src = edge_index[0]
    dst = edge_index[1]
    q = _linear(x_dst, p["q"])
    k = _linear(x_src, p["k"])
    v = _linear(x_src, p["v"])
    e = _linear(edge_attr, p["e"])
    kj = jnp.take(k, src, axis=0) + e
    vj = jnp.take(v, src, axis=0) + e
    qi = jnp.take(q, dst, axis=0)
    logits = jnp.sum(qi * kj, axis=-1) * INV_SQRT_HID
    n_dst = x_dst.shape[0]
    m = jax.ops.segment_max(logits, dst, num_segments=n_dst)
    m = jnp.where(jnp.isfinite(m), m, 0.0)
    ex = jnp.exp(logits - jnp.take(m, dst, axis=0))
    denom = jax.ops.segment_sum(ex, dst, num_segments=n_dst)
    alpha = ex / (jnp.take(denom, dst, axis=0) + 1e-16)
    agg = jax.ops.segment_sum(vj * alpha[:, None], dst, num_segments=n_dst)
    return agg + _linear(x_dst, p["s"])


def kernel(params, edge_attr_ub, edge_attr_bu, user_n_id, book_n_id,
           edge_index_ub, edge_index_bu, edge_label_index):
    x_user = jnp.take(params["user_emb"], user_n_id, axis=0)
    x_book = jnp.take(params["book_emb"], book_n_id, axis=0)
    b1 = jax.nn.relu(_transformer_conv(x_user, x_book, edge_index_ub, edge_attr_ub, params["conv1_ub"]))
    u1 = jax.nn.relu(_transformer_conv(x_book, x_user, edge_index_bu, edge_attr_bu, params["conv1_bu"]))
    b2 = _transformer_conv(u1, b1, edge_index_ub, edge_attr_ub, params["conv2_ub"])
    u2 = _transformer_conv(b1, u1, edge_index_bu, edge_attr_bu, params["conv2_bu"])
    row = edge_label_index[0]
    col = edge_label_index[1]
    z = jnp.concatenate([jnp.take(u2, row, axis=0), jnp.take(b2, col, axis=0)], axis=-1)
    return _decoder_mlp(z, params["dec1"], params["dec2"], params["dec3"], params["dec4"])
